# Initial kernel scaffold; baseline (speedup 1.0000x reference)
import jax
import jax.numpy as jnp
from jax.experimental import pallas as pl


def kernel(feat, target, centers):
    raise NotImplementedError("write your pallas kernel here")


# speedup vs baseline: 1.1211x; 1.1211x over previous
import functools

import jax
import jax.numpy as jnp
from jax import lax
from jax.experimental import pallas as pl
from jax.experimental.pallas import tpu as pltpu
from jax.experimental.pallas import tpu_sc as plsc

_LAMDA = 0.5
_NC = 2
_NS = 16
_NW = _NC * _NS
_L = 16


def _make_sc_kernel(batch, dim, num_classes):
    rows_per_w = batch // _NW
    chunk = 256
    n_chunks = rows_per_w // chunk

    mesh = plsc.VectorSubcoreMesh(
        core_axis_name="c", subcore_axis_name="s",
        num_cores=_NC, num_subcores=_NS)

    @functools.partial(
        pl.kernel,
        out_type=jax.ShapeDtypeStruct((_NW, _L), jnp.float32),
        mesh=mesh,
        scratch_types=[
            pltpu.VMEM((rows_per_w,), jnp.int32),
            pltpu.VMEM((chunk, dim), jnp.float32),
            pltpu.VMEM((chunk, dim), jnp.float32),
            pltpu.VMEM((_L,), jnp.float32),
            pltpu.SemaphoreType.DMA,
        ],
    )
    def sck(feat_hbm, tgt_hbm, cent_hbm, out_hbm, idx_v, gbuf, fbuf, acc_v,
            sem):
        wid = lax.axis_index("s") * _NC + lax.axis_index("c")
        base = wid * rows_per_w
        pltpu.sync_copy(tgt_hbm.at[pl.ds(base, rows_per_w)], idx_v)

        acc = jnp.zeros((_L,), jnp.float32)
        for c in range(n_chunks):
            gat = pltpu.async_copy(
                cent_hbm.at[idx_v.at[pl.ds(c * chunk, chunk)]], gbuf, sem)
            pltpu.sync_copy(feat_hbm.at[pl.ds(base + c * chunk, chunk)], fbuf)
            gat.wait()

            def row_body(i, a):
                for k in range(0, dim, _L):
                    d = gbuf[i, pl.ds(k, _L)] - fbuf[i, pl.ds(k, _L)]
                    a = a + d * d
                return a

            acc = lax.fori_loop(0, chunk, row_body, acc)

        acc_v[...] = acc
        pltpu.sync_copy(acc_v, out_hbm.at[wid])

    return sck


def kernel(feat, target, centers):
    batch, dim = feat.shape
    sck = _make_sc_kernel(batch, dim, centers.shape[0])
    partials = sck(feat, target.astype(jnp.int32), centers)
    return _LAMDA * jnp.sum(partials) / 2.0 / batch


# speedup vs baseline: 1.1702x; 1.0438x over previous
import functools

import jax
import jax.numpy as jnp
from jax import lax
from jax.experimental import pallas as pl
from jax.experimental.pallas import tpu as pltpu
from jax.experimental.pallas import tpu_sc as plsc

_LAMDA = 0.5
_NC = 2
_NS = 16
_NW = _NC * _NS
_L = 16


def _make_sc_kernel(batch, dim, num_classes):
    rows_per_w = batch // _NW
    chunk = 128
    n_chunks = rows_per_w // chunk
    n_acc = dim // _L

    mesh = plsc.VectorSubcoreMesh(
        core_axis_name="c", subcore_axis_name="s",
        num_cores=_NC, num_subcores=_NS)

    @functools.partial(
        pl.kernel,
        out_type=jax.ShapeDtypeStruct((_NW, _L), jnp.float32),
        mesh=mesh,
        scratch_types=[
            pltpu.VMEM((rows_per_w,), jnp.int32),
            pltpu.VMEM((chunk, dim), jnp.float32),
            pltpu.VMEM((chunk, dim), jnp.float32),
            pltpu.VMEM((rows_per_w, dim), jnp.float32),
            pltpu.VMEM((_L,), jnp.float32),
            pltpu.SemaphoreType.DMA,
            pltpu.SemaphoreType.DMA,
            pltpu.SemaphoreType.DMA,
        ],
    )
    def sck(feat_hbm, tgt_hbm, cent_hbm, out_hbm, idx_v, gbuf_a, gbuf_b,
            fbuf, acc_v, sem_a, sem_b, sem_f):
        wid = lax.axis_index("s") * _NC + lax.axis_index("c")
        base = wid * rows_per_w
        gbufs = (gbuf_a, gbuf_b)
        sems = (sem_a, sem_b)

        pltpu.sync_copy(tgt_hbm.at[pl.ds(base, rows_per_w)], idx_v)
        feat_cp = pltpu.async_copy(
            feat_hbm.at[pl.ds(base, rows_per_w)], fbuf, sem_f)
        copies = [None] * n_chunks

        def start_gather(c):
            copies[c] = pltpu.async_copy(
                cent_hbm.at[idx_v.at[pl.ds(c * chunk, chunk)]],
                gbufs[c % 2], sems[c % 2])

        start_gather(0)
        accs = tuple(jnp.zeros((_L,), jnp.float32) for _ in range(n_acc))
        for c in range(n_chunks):
            if c + 1 < n_chunks:
                start_gather(c + 1)
            copies[c].wait()
            if c == 0:
                feat_cp.wait()
            gbuf = gbufs[c % 2]
            foff = c * chunk

            def row_body(i, a):
                out = []
                for k in range(n_acc):
                    d = (gbuf[i, pl.ds(k * _L, _L)]
                         - fbuf[foff + i, pl.ds(k * _L, _L)])
                    out.append(a[k] + d * d)
                return tuple(out)

            accs = lax.fori_loop(0, chunk, row_body, accs)

        total = accs[0]
        for k in range(1, n_acc):
            total = total + accs[k]
        acc_v[...] = total
        pltpu.sync_copy(acc_v, out_hbm.at[wid])

    return sck


def kernel(feat, target, centers):
    batch, dim = feat.shape
    sck = _make_sc_kernel(batch, dim, centers.shape[0])
    partials = sck(feat, target.astype(jnp.int32), centers)
    return _LAMDA * jnp.sum(partials) / 2.0 / batch


# speedup vs baseline: 1.1720x; 1.0015x over previous
import functools

import jax
import jax.numpy as jnp
from jax import lax
from jax.experimental import pallas as pl
from jax.experimental.pallas import tpu as pltpu
from jax.experimental.pallas import tpu_sc as plsc

_LAMDA = 0.5
_NC = 2
_NS = 16
_NW = _NC * _NS
_L = 16


def _make_sc_kernel(batch, dim, num_classes):
    rows_per_w = batch // _NW
    chunk = 128
    n_chunks = rows_per_w // chunk
    n_acc = dim // _L

    mesh = plsc.VectorSubcoreMesh(
        core_axis_name="c", subcore_axis_name="s",
        num_cores=_NC, num_subcores=_NS)

    @functools.partial(
        pl.kernel,
        out_type=jax.ShapeDtypeStruct((_NW, _L), jnp.float32),
        mesh=mesh,
        scratch_types=[
            pltpu.VMEM((rows_per_w,), jnp.int32),
            pltpu.VMEM((chunk, dim), jnp.float32),
            pltpu.VMEM((chunk, dim), jnp.float32),
            pltpu.VMEM((rows_per_w, dim), jnp.float32),
            pltpu.VMEM((_L,), jnp.float32),
            pltpu.SemaphoreType.DMA,
            pltpu.SemaphoreType.DMA,
            pltpu.SemaphoreType.DMA,
        ],
    )
    def sck(feat_hbm, tgt_hbm, cent_hbm, out_hbm, idx_v, gbuf_a, gbuf_b,
            fbuf, acc_v, sem_a, sem_b, sem_f):
        wid = lax.axis_index("s") * _NC + lax.axis_index("c")
        base = wid * rows_per_w
        gbufs = (gbuf_a, gbuf_b)
        sems = (sem_a, sem_b)

        pltpu.sync_copy(tgt_hbm.at[pl.ds(base, rows_per_w)], idx_v)
        feat_cp = pltpu.async_copy(
            feat_hbm.at[pl.ds(base, rows_per_w)], fbuf, sem_f)
        copies = [None] * n_chunks

        def start_gather(c):
            copies[c] = pltpu.async_copy(
                cent_hbm.at[idx_v.at[pl.ds(c * chunk, chunk)]],
                gbufs[c % 2], sems[c % 2])

        start_gather(0)
        accs = tuple(jnp.zeros((_L,), jnp.float32) for _ in range(n_acc))
        for c in range(n_chunks):
            if c + 1 < n_chunks:
                start_gather(c + 1)
            copies[c].wait()
            if c == 0:
                feat_cp.wait()
            gbuf = gbufs[c % 2]
            foff = c * chunk

            @plsc.parallel_loop(0, chunk, carry=accs, unroll=4)
            def accs(i, a):
                out = []
                for k in range(n_acc):
                    d = (gbuf[i, pl.ds(k * _L, _L)]
                         - fbuf[foff + i, pl.ds(k * _L, _L)])
                    out.append(a[k] + d * d)
                return tuple(out)

        total = accs[0]
        for k in range(1, n_acc):
            total = total + accs[k]
        acc_v[...] = total
        pltpu.sync_copy(acc_v, out_hbm.at[wid])

    return sck


def kernel(feat, target, centers):
    batch, dim = feat.shape
    sck = _make_sc_kernel(batch, dim, centers.shape[0])
    partials = sck(feat, target.astype(jnp.int32), centers)
    return _LAMDA * jnp.sum(partials) / 2.0 / batch


# speedup vs baseline: 1.1798x; 1.0066x over previous
import functools

import jax
import jax.numpy as jnp
from jax import lax
from jax.experimental import pallas as pl
from jax.experimental.pallas import tpu as pltpu
from jax.experimental.pallas import tpu_sc as plsc

_LAMDA = 0.5
_NC = 2
_NS = 16
_NW = _NC * _NS
_L = 16


def _make_sc_kernel(batch, dim, num_classes):
    rows_per_w = batch // _NW
    chunk = 64
    n_chunks = rows_per_w // chunk
    n_buf = 3
    n_acc = dim // _L

    mesh = plsc.VectorSubcoreMesh(
        core_axis_name="c", subcore_axis_name="s",
        num_cores=_NC, num_subcores=_NS)

    @functools.partial(
        pl.kernel,
        out_type=jax.ShapeDtypeStruct((_NW, _L), jnp.float32),
        mesh=mesh,
        scratch_types=[
            pltpu.VMEM((rows_per_w,), jnp.int32),
            [pltpu.VMEM((chunk, dim), jnp.float32)] * n_buf,
            pltpu.VMEM((rows_per_w, dim), jnp.float32),
            pltpu.VMEM((_L,), jnp.float32),
            [pltpu.SemaphoreType.DMA] * n_buf,
            pltpu.SemaphoreType.DMA,
        ],
    )
    def sck(feat_hbm, tgt_hbm, cent_hbm, out_hbm, idx_v, gbufs, fbuf,
            acc_v, sems, sem_f):
        wid = lax.axis_index("s") * _NC + lax.axis_index("c")
        base = wid * rows_per_w
        copies = [None] * n_chunks

        def start_gather(c):
            copies[c] = pltpu.async_copy(
                cent_hbm.at[idx_v.at[pl.ds(c * chunk, chunk)]],
                gbufs[c % n_buf], sems[c % n_buf])

        pltpu.sync_copy(tgt_hbm.at[pl.ds(base, chunk)],
                        idx_v.at[pl.ds(0, chunk)])
        start_gather(0)
        feat_cp = pltpu.async_copy(
            feat_hbm.at[pl.ds(base, rows_per_w)], fbuf, sem_f)
        pltpu.sync_copy(tgt_hbm.at[pl.ds(base + chunk, rows_per_w - chunk)],
                        idx_v.at[pl.ds(chunk, rows_per_w - chunk)])
        for c in range(1, n_buf):
            start_gather(c)

        accs = tuple(jnp.zeros((_L,), jnp.float32) for _ in range(n_acc))
        for c in range(n_chunks):
            copies[c].wait()
            if c == 0:
                feat_cp.wait()
            gbuf = gbufs[c % n_buf]
            foff = c * chunk

            @plsc.parallel_loop(0, chunk, carry=accs, unroll=4)
            def accs(i, a):
                out = []
                for k in range(n_acc):
                    d = (gbuf[i, pl.ds(k * _L, _L)]
                         - fbuf[foff + i, pl.ds(k * _L, _L)])
                    out.append(a[k] + d * d)
                return tuple(out)

            if c + n_buf < n_chunks:
                start_gather(c + n_buf)

        total = accs[0]
        for k in range(1, n_acc):
            total = total + accs[k]
        acc_v[...] = total
        pltpu.sync_copy(acc_v, out_hbm.at[wid])

    return sck


def kernel(feat, target, centers):
    batch, dim = feat.shape
    sck = _make_sc_kernel(batch, dim, centers.shape[0])
    partials = sck(feat, target.astype(jnp.int32), centers)
    return _LAMDA * jnp.sum(partials) / 2.0 / batch


# speedup vs baseline: 1.2367x; 1.0482x over previous
import functools

import jax
import jax.numpy as jnp
from jax import lax
from jax.experimental import pallas as pl
from jax.experimental.pallas import tpu as pltpu
from jax.experimental.pallas import tpu_sc as plsc

_LAMDA = 0.5
_NC = 2
_NS = 16
_NW = _NC * _NS
_L = 16


def _make_sc_kernel(batch, dim, num_classes):
    rows_per_w = batch // _NW
    chunk = 64
    n_chunks = rows_per_w // chunk
    n_buf = 3
    n_acc = dim // _L

    mesh = plsc.VectorSubcoreMesh(
        core_axis_name="c", subcore_axis_name="s",
        num_cores=_NC, num_subcores=_NS)

    @functools.partial(
        pl.kernel,
        out_type=jax.ShapeDtypeStruct((_NW, _L), jnp.float32),
        mesh=mesh,
        scratch_types=[
            pltpu.VMEM((rows_per_w,), jnp.int32),
            [pltpu.VMEM((chunk, dim), jnp.float32)] * n_buf,
            [pltpu.VMEM((chunk, dim), jnp.float32)] * n_buf,
            pltpu.VMEM((_L,), jnp.float32),
            [pltpu.SemaphoreType.DMA] * n_buf,
            [pltpu.SemaphoreType.DMA] * n_buf,
        ],
    )
    def sck(feat_hbm, tgt_hbm, cent_hbm, out_hbm, idx_v, gbufs, fbufs,
            acc_v, gsems, fsems):
        wid = lax.axis_index("s") * _NC + lax.axis_index("c")
        base = wid * rows_per_w
        gcopies = [None] * n_chunks
        fcopies = [None] * n_chunks

        def start_chunk(c):
            gcopies[c] = pltpu.async_copy(
                cent_hbm.at[idx_v.at[pl.ds(c * chunk, chunk)]],
                gbufs[c % n_buf], gsems[c % n_buf])
            fcopies[c] = pltpu.async_copy(
                feat_hbm.at[pl.ds(base + c * chunk, chunk)],
                fbufs[c % n_buf], fsems[c % n_buf])

        pltpu.sync_copy(tgt_hbm.at[pl.ds(base, chunk)],
                        idx_v.at[pl.ds(0, chunk)])
        start_chunk(0)
        pltpu.sync_copy(tgt_hbm.at[pl.ds(base + chunk, rows_per_w - chunk)],
                        idx_v.at[pl.ds(chunk, rows_per_w - chunk)])
        for c in range(1, n_buf):
            start_chunk(c)

        accs = tuple(jnp.zeros((_L,), jnp.float32) for _ in range(n_acc))
        for c in range(n_chunks):
            gcopies[c].wait()
            fcopies[c].wait()
            gbuf = gbufs[c % n_buf]
            fbuf = fbufs[c % n_buf]

            @plsc.parallel_loop(0, chunk, carry=accs, unroll=4)
            def accs(i, a):
                out = []
                for k in range(n_acc):
                    d = (gbuf[i, pl.ds(k * _L, _L)]
                         - fbuf[i, pl.ds(k * _L, _L)])
                    out.append(a[k] + d * d)
                return tuple(out)

            if c + n_buf < n_chunks:
                start_chunk(c + n_buf)

        total = accs[0]
        for k in range(1, n_acc):
            total = total + accs[k]
        acc_v[...] = total
        pltpu.sync_copy(acc_v, out_hbm.at[wid])

    return sck


def kernel(feat, target, centers):
    batch, dim = feat.shape
    sck = _make_sc_kernel(batch, dim, centers.shape[0])
    partials = sck(feat, target.astype(jnp.int32), centers)
    return _LAMDA * jnp.sum(partials) / 2.0 / batch
